# Initial kernel scaffold; baseline (speedup 1.0000x reference)
import jax
import jax.numpy as jnp
from jax.experimental import pallas as pl


def kernel(cell_x, well_x, c2c_edge_index, c2c_edge_attr, c2w_src, c2w_dst, cell_emb_w, cell_emb_b, well_emb_w, well_emb_b, edge_emb_w, edge_emb_b, L_lin_w, L_att_src, L_att_dst, L_lin_edge_w, L_att_edge, L_gat_bias, L_n1_g, L_n1_b, L_ffn_w1, L_ffn_b1, L_ffn_w2, L_ffn_b2, L_n2_g, L_n2_b, wc_lin_w, wc_att_src, wc_att_dst, wc_bias, mlp_w1, mlp_b1, mlp_w2, mlp_b2):
    raise NotImplementedError("write your pallas kernel here")


# speedup vs baseline: 10.8218x; 10.8218x over previous
import functools

import jax
import jax.numpy as jnp
from jax import lax
from jax.experimental import pallas as pl
from jax.experimental.pallas import tpu as pltpu
from jax.experimental.pallas import tpu_sc as plsc

N_CELL = 10000
N_WELL = 500
E_C2C = 320000
E_C2W = 25000
H = 128
NL = 3
FF = 512
OUT = 75

NP = 10240
WP = 512
NW = 32
EP1 = NW * 10240
EP2 = NW * 896

_NEG = -1e30


def _ln(x, g, b, eps=1e-5):
    m = x.mean(-1, keepdims=True)
    v = ((x - m) ** 2).mean(-1, keepdims=True)
    return (x - m) * jax.lax.rsqrt(v + eps) * g + b


def _t0_body(cx, wx, attr, cw, cb, ww, wb, ew, eb, lin_e, att_e,
             hc_o, hw_o, et_o, etmax_o):
    hc_o[...] = cx[...] @ cw[...] + cb[...]
    hw_o[...] = wx[...] @ ww[...] + wb[...]
    a = attr[...]
    mx = jnp.max(a)
    mn = jnp.min(a)
    for l in range(NL):
        c1 = jnp.sum((ew[...][0] @ lin_e[...][l]) * att_e[...][l])
        c0 = jnp.sum((eb[...] @ lin_e[...][l]) * att_e[...][l])
        et_o[l, :] = a * c1 + c0
        m = jnp.maximum(jnp.abs(c1 * mx + c0), jnp.abs(c1 * mn + c0))
        etmax_o[l, :] = jnp.full((H,), m, jnp.float32)


def _t1_body(h, g1, b1, w, att_s, att_d, etmax, xs_o, as_o, ad_o, m_o):
    xn = _ln(h[...], g1[...], b1[...])
    xs = xn @ w[...]
    xs_o[...] = xs
    asf = jnp.sum(xs * att_s[...], axis=-1)
    adf = jnp.sum(xs * att_d[...], axis=-1)
    rows = lax.broadcasted_iota(jnp.int32, (NP,), 0)
    mask = rows < N_CELL
    as_o[...] = jnp.where(mask, asf, 0.0)
    ad_o[...] = jnp.where(mask, adf, 0.0)
    m = jnp.maximum(
        jnp.max(jnp.where(mask, asf, _NEG))
        + jnp.max(jnp.where(mask, adf, _NEG))
        + etmax[...][0], 0.0)
    m_o[...] = jnp.full((H,), m, jnp.float32)


def _t2_body(h, msg0, msg1, den0, den1, gbias, g2, b2, w1, bb1, w2, bb2,
             h_o):
    den = (den0[...] + den1[...] + 1e-16)[:, None]
    h1 = h[...] + (msg0[...] + msg1[...]) / den + gbias[...]
    xn = _ln(h1, g2[...], b2[...])
    h_o[...] = h1 + jax.nn.relu(xn @ w1[...] + bb1[...]) @ w2[...] + bb2[...]


def _t3_body(h, hw, w, att_s, att_d, xs_o, as_o, ad_o, m_o):
    xs = h[...] @ w[...]
    xs_o[...] = xs
    xd = hw[...] @ w[...]
    asf = jnp.sum(xs * att_s[...], axis=-1)
    adf = jnp.sum(xd * att_d[...], axis=-1)
    rc = lax.broadcasted_iota(jnp.int32, (NP,), 0)
    rw = lax.broadcasted_iota(jnp.int32, (WP,), 0)
    mc = rc < N_CELL
    mw = rw < N_WELL
    as_o[...] = jnp.where(mc, asf, 0.0)
    ad_o[...] = jnp.where(mw, adf, 0.0)
    m = jnp.maximum(jnp.max(jnp.where(mc, asf, _NEG))
                    + jnp.max(jnp.where(mw, adf, _NEG)), 0.0)
    m_o[...] = jnp.full((H,), m, jnp.float32)


def _t4_body(msg0, msg1, den0, den1, bias, w1, b1, w2, b2, out_o):
    den = (den0[...] + den1[...] + 1e-16)[:, None]
    hw = (msg0[...] + msg1[...]) / den + bias[...]
    out_o[...] = jax.nn.relu(hw @ w1[...] + b1[...]) @ w2[...] + b2[...]


def _tc(body, out_shapes):
    return pl.pallas_call(body, out_shape=out_shapes)


def _make_edge_kernel(nsrc, ndst, ep):
    slab = ep // NW
    ch = slab // 128
    r16 = ndst // 16
    mesh = plsc.VectorSubcoreMesh(core_axis_name="c", subcore_axis_name="s")

    @functools.partial(
        pl.kernel, mesh=mesh,
        compiler_params=pltpu.CompilerParams(needs_layout_passes=False),
        out_type=(jax.ShapeDtypeStruct((2, ndst, H), jnp.float32),
                  jax.ShapeDtypeStruct((2, ndst), jnp.float32)),
        scratch_types=[
            pltpu.VMEM((128,), jnp.int32),
            pltpu.VMEM((1, 128), jnp.int32),
            pltpu.VMEM((128,), jnp.float32),
            pltpu.VMEM((128,), jnp.float32),
            pltpu.VMEM((128,), jnp.float32),
            pltpu.VMEM((128, H), jnp.float32),
            pltpu.VMEM((16,), jnp.float32),
            pltpu.VMEM_SHARED((ndst,), jnp.float32),
            pltpu.VMEM_SHARED((ndst, H), jnp.float32),
            pltpu.SemaphoreType.DMA,
        ],
    )
    def edge_kernel(xs_hbm, asrc_hbm, adst_hbm, src_hbm, dst2_hbm, et_hbm,
                    m_hbm, out_hbm, den_hbm, idx_s, idx_d, et_c, as_c, ad_c,
                    rows, m_v, sh_den, sh_msg, sem):
        c = lax.axis_index("c")
        t = lax.axis_index("s")
        w = 2 * t + c

        def _zrow(r, _):
            for f in range(H // 16):
                rows[r, pl.ds(f * 16, 16)] = jnp.zeros((16,), jnp.float32)
            return 0
        lax.fori_loop(0, 128, _zrow, 0)

        r0 = t * r16
        nfull, rem = divmod(r16, 128)
        for k in range(nfull):
            pltpu.sync_copy(rows, sh_msg.at[pl.ds(r0 + k * 128, 128), :])
        if rem:
            pltpu.sync_copy(rows.at[pl.ds(0, rem), :],
                            sh_msg.at[pl.ds(r0 + nfull * 128, rem), :])
        nsl = ndst // 128
        for k in range((nsl + 15) // 16):
            i = t + 16 * k

            @pl.when(i < nsl)
            def _zden():
                pltpu.sync_copy(rows.at[0], sh_den.at[pl.ds(i * 128, 128)])

        pltpu.sync_copy(m_hbm.at[pl.ds(0, 16)], m_v)
        plsc.subcore_barrier()
        mb = m_v[...]

        def _chunk(cc, _):
            e0 = w * slab + cc * 128
            pltpu.sync_copy(src_hbm.at[pl.ds(e0, 128)], idx_s)
            pltpu.sync_copy(dst2_hbm.at[w, cc], idx_d.at[0])
            pltpu.sync_copy(et_hbm.at[pl.ds(e0, 128)], et_c)
            pltpu.async_copy(asrc_hbm.at[idx_s], as_c, sem).wait()
            pltpu.async_copy(adst_hbm.at[idx_d.at[0]], ad_c, sem).wait()
            for k in range(8):
                sl = pl.ds(k * 16, 16)
                z = as_c[sl] + ad_c[sl] + et_c[sl]
                a = jnp.where(z >= 0.0, z, 0.2 * z)
                et_c[sl] = jnp.exp(a - mb)
            pltpu.sync_copy(et_c, sh_den.at[idx_d.at[0]], add=True)
            pltpu.async_copy(xs_hbm.at[idx_s], rows, sem).wait()

            def _scale(r, _):
                cb = plsc.load_gather(et_c, [jnp.full((16,), r, jnp.int32)])
                for f in range(H // 16):
                    fl = pl.ds(f * 16, 16)
                    rows[r, fl] = rows[r, fl] * cb
                return 0
            lax.fori_loop(0, 128, _scale, 0)
            pltpu.sync_copy(rows, sh_msg.at[idx_d.at[0]], add=True)
            return 0
        lax.fori_loop(0, ch, _chunk, 0)

        plsc.subcore_barrier()
        pltpu.sync_copy(sh_msg.at[pl.ds(r0, r16), :],
                        out_hbm.at[c, pl.ds(r0, r16), :])
        for k in range((nsl + 15) // 16):
            i = t + 16 * k

            @pl.when(i < nsl)
            def _cden():
                pltpu.sync_copy(sh_den.at[pl.ds(i * 128, 128)],
                                den_hbm.at[c, pl.ds(i * 128, 128)])

    return edge_kernel


_edge_c2c = _make_edge_kernel(NP, NP, EP1)
_edge_c2w = _make_edge_kernel(NP, WP, EP2)


def kernel(cell_x, well_x, c2c_edge_index, c2c_edge_attr, c2w_src, c2w_dst,
           cell_emb_w, cell_emb_b, well_emb_w, well_emb_b, edge_emb_w,
           edge_emb_b, L_lin_w, L_att_src, L_att_dst, L_lin_edge_w,
           L_att_edge, L_gat_bias, L_n1_g, L_n1_b, L_ffn_w1, L_ffn_b1,
           L_ffn_w2, L_ffn_b2, L_n2_g, L_n2_b, wc_lin_w, wc_att_src,
           wc_att_dst, wc_bias, mlp_w1, mlp_b1, mlp_w2, mlp_b2):
    f32 = jnp.float32
    cxp = jnp.zeros((NP, H), f32).at[:N_CELL].set(cell_x)
    wxp = jnp.zeros((WP, well_x.shape[1]), f32).at[:N_WELL].set(well_x)
    attr = c2c_edge_attr[:, 0]

    src1 = jnp.zeros((EP1,), jnp.int32).at[:E_C2C].set(
        c2c_edge_index[0].astype(jnp.int32))
    dst1 = jnp.zeros((EP1,), jnp.int32).at[:E_C2C].set(
        c2c_edge_index[1].astype(jnp.int32))
    dst1_2 = dst1.reshape(NW, EP1 // NW // 128, 128)
    src2 = jnp.zeros((EP2,), jnp.int32).at[:E_C2W].set(
        c2w_src.astype(jnp.int32))
    dst2 = jnp.zeros((EP2,), jnp.int32).at[:E_C2W].set(
        c2w_dst.astype(jnp.int32))
    dst2_2 = dst2.reshape(NW, EP2 // NW // 128, 128)

    h_cell, h_well, et3, etmax3 = _tc(
        _t0_body,
        (jax.ShapeDtypeStruct((NP, H), f32),
         jax.ShapeDtypeStruct((WP, H), f32),
         jax.ShapeDtypeStruct((NL, E_C2C), f32),
         jax.ShapeDtypeStruct((NL, H), f32)))(
        cxp, wxp, attr, cell_emb_w, cell_emb_b, well_emb_w, well_emb_b,
        edge_emb_w, edge_emb_b, L_lin_edge_w, L_att_edge)

    pad_neg = jnp.full((EP1 - E_C2C,), _NEG, f32)
    for l in range(NL):
        xs, a_s, a_d, m = _tc(
            _t1_body,
            (jax.ShapeDtypeStruct((NP, H), f32),
             jax.ShapeDtypeStruct((NP,), f32),
             jax.ShapeDtypeStruct((NP,), f32),
             jax.ShapeDtypeStruct((H,), f32)))(
            h_cell, L_n1_g[l], L_n1_b[l], L_lin_w[l], L_att_src[l],
            L_att_dst[l], etmax3[l])
        et_l = jnp.concatenate([et3[l], pad_neg])
        msg, den = _edge_c2c(xs, a_s, a_d, src1, dst1_2, et_l, m)
        h_cell = _tc(
            _t2_body, jax.ShapeDtypeStruct((NP, H), f32))(
            h_cell, msg[0], msg[1], den[0], den[1], L_gat_bias[l],
            L_n2_g[l], L_n2_b[l], L_ffn_w1[l], L_ffn_b1[l], L_ffn_w2[l],
            L_ffn_b2[l])

    xsw, asw, adw, mw = _tc(
        _t3_body,
        (jax.ShapeDtypeStruct((NP, H), f32),
         jax.ShapeDtypeStruct((NP,), f32),
         jax.ShapeDtypeStruct((WP,), f32),
         jax.ShapeDtypeStruct((H,), f32)))(
        h_cell, h_well, wc_lin_w, wc_att_src, wc_att_dst)
    etw = jnp.concatenate([jnp.zeros((E_C2W,), f32),
                           jnp.full((EP2 - E_C2W,), _NEG, f32)])
    msgw, denw = _edge_c2w(xsw, asw, adw, src2, dst2_2, etw, mw)

    w2p = jnp.zeros((H, H), f32).at[:, :OUT].set(mlp_w2)
    b2p = jnp.zeros((H,), f32).at[:OUT].set(mlp_b2)
    out = _tc(_t4_body, jax.ShapeDtypeStruct((WP, H), f32))(
        msgw[0], msgw[1], denw[0], denw[1], wc_bias, mlp_w1, mlp_b1,
        w2p, b2p)
    return out[:N_WELL, :OUT].reshape(N_WELL, 3, 25)


# speedup vs baseline: 14.9761x; 1.3839x over previous
import functools

import jax
import jax.numpy as jnp
from jax import lax
from jax.experimental import pallas as pl
from jax.experimental.pallas import tpu as pltpu
from jax.experimental.pallas import tpu_sc as plsc

N_CELL = 10000
N_WELL = 500
E_C2C = 320000
E_C2W = 25000
H = 128
NL = 3
FF = 512
OUT = 75

NP = 10240
WP = 512
NW = 32
EP1 = NW * 10240
EP2 = NW * 896

_NEG = -1e30


def _ln(x, g, b, eps=1e-5):
    m = x.mean(-1, keepdims=True)
    v = ((x - m) ** 2).mean(-1, keepdims=True)
    return (x - m) * jax.lax.rsqrt(v + eps) * g + b


def _t0_body(cx, wx, attr, cw, cb, ww, wb, ew, eb, lin_e, att_e,
             hc_o, hw_o, et_o, etmax_o):
    hc_o[...] = cx[...] @ cw[...] + cb[...]
    hw_o[...] = wx[...] @ ww[...] + wb[...]
    a = attr[...]
    mx = jnp.max(a)
    mn = jnp.min(a)
    for l in range(NL):
        c1 = jnp.sum((ew[...][0] @ lin_e[...][l]) * att_e[...][l])
        c0 = jnp.sum((eb[...] @ lin_e[...][l]) * att_e[...][l])
        et_o[l, :] = a * c1 + c0
        m = jnp.maximum(jnp.abs(c1 * mx + c0), jnp.abs(c1 * mn + c0))
        etmax_o[l, :] = jnp.full((H,), m, jnp.float32)


def _t1_body(h, g1, b1, w, att_s, att_d, etmax, xs_o, as_o, ad_o, m_o):
    xn = _ln(h[...], g1[...], b1[...])
    xs = xn @ w[...]
    xs_o[...] = xs
    asf = jnp.sum(xs * att_s[...], axis=-1)
    adf = jnp.sum(xs * att_d[...], axis=-1)
    rows = lax.broadcasted_iota(jnp.int32, (NP,), 0)
    mask = rows < N_CELL
    as_o[...] = jnp.where(mask, asf, 0.0)
    ad_o[...] = jnp.where(mask, adf, 0.0)
    m = jnp.maximum(
        jnp.max(jnp.where(mask, asf, _NEG))
        + jnp.max(jnp.where(mask, adf, _NEG))
        + etmax[...][0], 0.0)
    m_o[...] = jnp.full((H,), m, jnp.float32)


def _t2_body(h, msg0, msg1, den0, den1, gbias, g2, b2, w1, bb1, w2, bb2,
             h_o):
    den = (den0[...] + den1[...] + 1e-16)[:, None]
    h1 = h[...] + (msg0[...] + msg1[...]) / den + gbias[...]
    xn = _ln(h1, g2[...], b2[...])
    h_o[...] = h1 + jax.nn.relu(xn @ w1[...] + bb1[...]) @ w2[...] + bb2[...]


def _t3_body(h, hw, w, att_s, att_d, xs_o, as_o, ad_o, m_o):
    xs = h[...] @ w[...]
    xs_o[...] = xs
    xd = hw[...] @ w[...]
    asf = jnp.sum(xs * att_s[...], axis=-1)
    adf = jnp.sum(xd * att_d[...], axis=-1)
    rc = lax.broadcasted_iota(jnp.int32, (NP,), 0)
    rw = lax.broadcasted_iota(jnp.int32, (WP,), 0)
    mc = rc < N_CELL
    mw = rw < N_WELL
    as_o[...] = jnp.where(mc, asf, 0.0)
    ad_o[...] = jnp.where(mw, adf, 0.0)
    m = jnp.maximum(jnp.max(jnp.where(mc, asf, _NEG))
                    + jnp.max(jnp.where(mw, adf, _NEG)), 0.0)
    m_o[...] = jnp.full((H,), m, jnp.float32)


def _t4_body(msg0, msg1, den0, den1, bias, w1, b1, w2, b2, out_o):
    den = (den0[...] + den1[...] + 1e-16)[:, None]
    hw = (msg0[...] + msg1[...]) / den + bias[...]
    out_o[...] = jax.nn.relu(hw @ w1[...] + b1[...]) @ w2[...] + b2[...]


def _tc(body, out_shapes):
    return pl.pallas_call(body, out_shape=out_shapes)


def _make_edge_kernel(nsrc, ndst, ep, bs):
    slab = ep // NW
    ch = slab // 128
    nb = ch // bs
    assert ch % bs == 0
    r16 = ndst // 16
    mesh = plsc.VectorSubcoreMesh(core_axis_name="c", subcore_axis_name="s")

    @functools.partial(
        pl.kernel, mesh=mesh,
        compiler_params=pltpu.CompilerParams(needs_layout_passes=False),
        out_type=(jax.ShapeDtypeStruct((2, ndst, H), jnp.float32),
                  jax.ShapeDtypeStruct((2, ndst), jnp.float32)),
        scratch_types=[
            pltpu.VMEM((bs * 128,), jnp.int32),
            pltpu.VMEM((bs, 128), jnp.int32),
            pltpu.VMEM((bs * 128,), jnp.float32),
            pltpu.VMEM((bs * 128,), jnp.float32),
            pltpu.VMEM((bs * 128,), jnp.float32),
            pltpu.VMEM((2, 128, H), jnp.float32),
            pltpu.VMEM((16,), jnp.float32),
            pltpu.VMEM_SHARED((ndst,), jnp.float32),
            pltpu.VMEM_SHARED((ndst, H), jnp.float32),
            pltpu.SemaphoreType.DMA,
            pltpu.SemaphoreType.DMA,
            pltpu.SemaphoreType.DMA,
        ],
    )
    def edge_kernel(xs_hbm, asrc_hbm, adst_hbm, src_hbm, dst2_hbm, et_hbm,
                    m_hbm, out_hbm, den_hbm, idx_s, idx_d, et_c, as_c, ad_c,
                    rows, m_v, sh_den, sh_msg, sem, gsem, dsem):
        c = lax.axis_index("c")
        t = lax.axis_index("s")
        w = 2 * t + c

        def _zrow(r, _):
            for f in range(H // 16):
                rows[0, r, pl.ds(f * 16, 16)] = jnp.zeros((16,), jnp.float32)
            return 0
        lax.fori_loop(0, 128, _zrow, 0)

        r0 = t * r16
        nfull, rem = divmod(r16, 128)
        for k in range(nfull):
            pltpu.sync_copy(rows.at[0],
                            sh_msg.at[pl.ds(r0 + k * 128, 128), :])
        if rem:
            pltpu.sync_copy(rows.at[0, pl.ds(0, rem), :],
                            sh_msg.at[pl.ds(r0 + nfull * 128, rem), :])
        nsl = ndst // 128
        for k in range((nsl + 15) // 16):
            i = t + 16 * k

            @pl.when(i < nsl)
            def _zden():
                pltpu.sync_copy(rows.at[0, 0],
                                sh_den.at[pl.ds(i * 128, 128)])

        pltpu.sync_copy(m_hbm.at[pl.ds(0, 16)], m_v)
        plsc.subcore_barrier()
        mb = m_v[...]

        def _batch(b, _):
            e0 = w * slab + b * (bs * 128)
            pltpu.sync_copy(src_hbm.at[pl.ds(e0, bs * 128)], idx_s)
            pltpu.sync_copy(dst2_hbm.at[w, pl.ds(b * bs, bs)], idx_d)
            pltpu.sync_copy(et_hbm.at[pl.ds(e0, bs * 128)], et_c)
            sc_d = []
            for cc in range(bs):
                sl = pl.ds(cc * 128, 128)
                sc_d.append(pltpu.async_copy(
                    asrc_hbm.at[idx_s.at[sl]], as_c.at[sl], sem))
                sc_d.append(pltpu.async_copy(
                    adst_hbm.at[idx_d.at[cc]], ad_c.at[sl], sem))
            for d in sc_d:
                d.wait()
            g = [None, None]
            g[0] = pltpu.async_copy(
                xs_hbm.at[idx_s.at[pl.ds(0, 128)]], rows.at[0], gsem)
            if bs > 1:
                g[1] = pltpu.async_copy(
                    xs_hbm.at[idx_s.at[pl.ds(128, 128)]], rows.at[1], gsem)

            def _exp(i, _):
                sl = pl.ds(i * 16, 16)
                z = as_c[sl] + ad_c[sl] + et_c[sl]
                a = jnp.where(z >= 0.0, z, 0.2 * z)
                et_c[sl] = jnp.exp(a - mb)
                return 0
            lax.fori_loop(0, bs * 8, _exp, 0)

            den_d = [pltpu.async_copy(et_c.at[pl.ds(cc * 128, 128)],
                                      sh_den.at[idx_d.at[cc]], dsem,
                                      add=True)
                     for cc in range(bs)]

            for cc in range(bs):
                pb = cc % 2
                g[pb].wait()

                def _scale(r, _, cc=cc, pb=pb):
                    cb = plsc.load_gather(
                        et_c, [jnp.full((16,), cc * 128 + r, jnp.int32)])
                    for f in range(H // 16):
                        fl = pl.ds(f * 16, 16)
                        rows[pb, r, fl] = rows[pb, r, fl] * cb
                    return 0
                lax.fori_loop(0, 128, _scale, 0)
                pltpu.sync_copy(rows.at[pb], sh_msg.at[idx_d.at[cc]],
                                add=True)
                nxt = cc + 2
                if nxt < bs:
                    g[pb] = pltpu.async_copy(
                        xs_hbm.at[idx_s.at[pl.ds(nxt * 128, 128)]],
                        rows.at[pb], gsem)
            for d in den_d:
                d.wait()
            return 0
        lax.fori_loop(0, nb, _batch, 0)

        plsc.subcore_barrier()
        pltpu.sync_copy(sh_msg.at[pl.ds(r0, r16), :],
                        out_hbm.at[c, pl.ds(r0, r16), :])
        for k in range((nsl + 15) // 16):
            i = t + 16 * k

            @pl.when(i < nsl)
            def _cden():
                pltpu.sync_copy(sh_den.at[pl.ds(i * 128, 128)],
                                den_hbm.at[c, pl.ds(i * 128, 128)])

    return edge_kernel


_edge_c2c = _make_edge_kernel(NP, NP, EP1, 4)
_edge_c2w = _make_edge_kernel(NP, WP, EP2, 1)


def kernel(cell_x, well_x, c2c_edge_index, c2c_edge_attr, c2w_src, c2w_dst,
           cell_emb_w, cell_emb_b, well_emb_w, well_emb_b, edge_emb_w,
           edge_emb_b, L_lin_w, L_att_src, L_att_dst, L_lin_edge_w,
           L_att_edge, L_gat_bias, L_n1_g, L_n1_b, L_ffn_w1, L_ffn_b1,
           L_ffn_w2, L_ffn_b2, L_n2_g, L_n2_b, wc_lin_w, wc_att_src,
           wc_att_dst, wc_bias, mlp_w1, mlp_b1, mlp_w2, mlp_b2):
    f32 = jnp.float32
    cxp = jnp.zeros((NP, H), f32).at[:N_CELL].set(cell_x)
    wxp = jnp.zeros((WP, well_x.shape[1]), f32).at[:N_WELL].set(well_x)
    attr = c2c_edge_attr[:, 0]

    src1 = jnp.zeros((EP1,), jnp.int32).at[:E_C2C].set(
        c2c_edge_index[0].astype(jnp.int32))
    dst1 = jnp.zeros((EP1,), jnp.int32).at[:E_C2C].set(
        c2c_edge_index[1].astype(jnp.int32))
    dst1_2 = dst1.reshape(NW, EP1 // NW // 128, 128)
    src2 = jnp.zeros((EP2,), jnp.int32).at[:E_C2W].set(
        c2w_src.astype(jnp.int32))
    dst2 = jnp.zeros((EP2,), jnp.int32).at[:E_C2W].set(
        c2w_dst.astype(jnp.int32))
    dst2_2 = dst2.reshape(NW, EP2 // NW // 128, 128)

    h_cell, h_well, et3, etmax3 = _tc(
        _t0_body,
        (jax.ShapeDtypeStruct((NP, H), f32),
         jax.ShapeDtypeStruct((WP, H), f32),
         jax.ShapeDtypeStruct((NL, E_C2C), f32),
         jax.ShapeDtypeStruct((NL, H), f32)))(
        cxp, wxp, attr, cell_emb_w, cell_emb_b, well_emb_w, well_emb_b,
        edge_emb_w, edge_emb_b, L_lin_edge_w, L_att_edge)

    pad_neg = jnp.full((EP1 - E_C2C,), _NEG, f32)
    for l in range(NL):
        xs, a_s, a_d, m = _tc(
            _t1_body,
            (jax.ShapeDtypeStruct((NP, H), f32),
             jax.ShapeDtypeStruct((NP,), f32),
             jax.ShapeDtypeStruct((NP,), f32),
             jax.ShapeDtypeStruct((H,), f32)))(
            h_cell, L_n1_g[l], L_n1_b[l], L_lin_w[l], L_att_src[l],
            L_att_dst[l], etmax3[l])
        et_l = jnp.concatenate([et3[l], pad_neg])
        msg, den = _edge_c2c(xs, a_s, a_d, src1, dst1_2, et_l, m)
        h_cell = _tc(
            _t2_body, jax.ShapeDtypeStruct((NP, H), f32))(
            h_cell, msg[0], msg[1], den[0], den[1], L_gat_bias[l],
            L_n2_g[l], L_n2_b[l], L_ffn_w1[l], L_ffn_b1[l], L_ffn_w2[l],
            L_ffn_b2[l])

    xsw, asw, adw, mw = _tc(
        _t3_body,
        (jax.ShapeDtypeStruct((NP, H), f32),
         jax.ShapeDtypeStruct((NP,), f32),
         jax.ShapeDtypeStruct((WP,), f32),
         jax.ShapeDtypeStruct((H,), f32)))(
        h_cell, h_well, wc_lin_w, wc_att_src, wc_att_dst)
    etw = jnp.concatenate([jnp.zeros((E_C2W,), f32),
                           jnp.full((EP2 - E_C2W,), _NEG, f32)])
    msgw, denw = _edge_c2w(xsw, asw, adw, src2, dst2_2, etw, mw)

    w2p = jnp.zeros((H, H), f32).at[:, :OUT].set(mlp_w2)
    b2p = jnp.zeros((H,), f32).at[:OUT].set(mlp_b2)
    out = _tc(_t4_body, jax.ShapeDtypeStruct((WP, H), f32))(
        msgw[0], msgw[1], denw[0], denw[1], wc_bias, mlp_w1, mlp_b1,
        w2p, b2p)
    return out[:N_WELL, :OUT].reshape(N_WELL, 3, 25)


# speedup vs baseline: 15.6926x; 1.0478x over previous
import functools

import jax
import jax.numpy as jnp
from jax import lax
from jax.experimental import pallas as pl
from jax.experimental.pallas import tpu as pltpu
from jax.experimental.pallas import tpu_sc as plsc

N_CELL = 10000
N_WELL = 500
E_C2C = 320000
E_C2W = 25000
H = 128
NL = 3
FF = 512
OUT = 75

NP = 10240
WP = 512
NW = 32
EP1 = NW * 10240
EP2 = NW * 896

_NEG = -1e30


def _ln(x, g, b, eps=1e-5):
    m = x.mean(-1, keepdims=True)
    v = ((x - m) ** 2).mean(-1, keepdims=True)
    return (x - m) * jax.lax.rsqrt(v + eps) * g + b


def _t0_body(cx, wx, attr, cw, cb, ww, wb, ew, eb, lin_e, att_e,
             hc_o, hw_o, et_o, etmax_o):
    hc_o[...] = cx[...] @ cw[...] + cb[...]
    hw_o[...] = wx[...] @ ww[...] + wb[...]
    a = attr[...]
    mx = jnp.max(a)
    mn = jnp.min(a)
    for l in range(NL):
        c1 = jnp.sum((ew[...][0] @ lin_e[...][l]) * att_e[...][l])
        c0 = jnp.sum((eb[...] @ lin_e[...][l]) * att_e[...][l])
        et_o[l, :] = a * c1 + c0
        m = jnp.maximum(jnp.abs(c1 * mx + c0), jnp.abs(c1 * mn + c0))
        etmax_o[l, :] = jnp.full((H,), m, jnp.float32)


def _t1_body(h, g1, b1, w, att_s, att_d, etmax, xs_o, as_o, ad_o, m_o):
    xn = _ln(h[...], g1[...], b1[...])
    xs = xn @ w[...]
    xs_o[...] = xs
    asf = jnp.sum(xs * att_s[...], axis=-1)
    adf = jnp.sum(xs * att_d[...], axis=-1)
    rows = lax.broadcasted_iota(jnp.int32, (NP,), 0)
    mask = rows < N_CELL
    as_o[...] = jnp.where(mask, asf, 0.0)
    ad_o[...] = jnp.where(mask, adf, 0.0)
    m = jnp.maximum(
        jnp.max(jnp.where(mask, asf, _NEG))
        + jnp.max(jnp.where(mask, adf, _NEG))
        + etmax[...][0], 0.0)
    m_o[...] = jnp.full((H,), m, jnp.float32)


def _t2_body(h, msg0, msg1, den0, den1, gbias, g2, b2, w1, bb1, w2, bb2,
             h_o):
    den = (den0[...] + den1[...] + 1e-16)[:, None]
    h1 = h[...] + (msg0[...] + msg1[...]) / den + gbias[...]
    xn = _ln(h1, g2[...], b2[...])
    h_o[...] = h1 + jax.nn.relu(xn @ w1[...] + bb1[...]) @ w2[...] + bb2[...]


def _t3_body(h, hw, w, att_s, att_d, xs_o, as_o, ad_o, m_o):
    xs = h[...] @ w[...]
    xs_o[...] = xs
    xd = hw[...] @ w[...]
    asf = jnp.sum(xs * att_s[...], axis=-1)
    adf = jnp.sum(xd * att_d[...], axis=-1)
    rc = lax.broadcasted_iota(jnp.int32, (NP,), 0)
    rw = lax.broadcasted_iota(jnp.int32, (WP,), 0)
    mc = rc < N_CELL
    mw = rw < N_WELL
    as_o[...] = jnp.where(mc, asf, 0.0)
    ad_o[...] = jnp.where(mw, adf, 0.0)
    m = jnp.maximum(jnp.max(jnp.where(mc, asf, _NEG))
                    + jnp.max(jnp.where(mw, adf, _NEG)), 0.0)
    m_o[...] = jnp.full((H,), m, jnp.float32)


def _t4_body(msg0, msg1, den0, den1, bias, w1, b1, w2, b2, out_o):
    den = (den0[...] + den1[...] + 1e-16)[:, None]
    hw = (msg0[...] + msg1[...]) / den + bias[...]
    out_o[...] = jax.nn.relu(hw @ w1[...] + b1[...]) @ w2[...] + b2[...]


def _tc(body, out_shapes):
    return pl.pallas_call(body, out_shape=out_shapes)


def _make_edge_kernel(nsrc, ndst, ep, bs):
    slab = ep // NW
    ch = slab // 128
    nb = ch // bs
    assert ch % bs == 0
    r16 = ndst // 16
    mesh = plsc.VectorSubcoreMesh(core_axis_name="c", subcore_axis_name="s")

    @functools.partial(
        pl.kernel, mesh=mesh,
        compiler_params=pltpu.CompilerParams(needs_layout_passes=False),
        out_type=(jax.ShapeDtypeStruct((2, ndst, H), jnp.float32),
                  jax.ShapeDtypeStruct((2, ndst), jnp.float32)),
        scratch_types=[
            pltpu.VMEM((bs * 128,), jnp.int32),
            pltpu.VMEM((bs, 128), jnp.int32),
            pltpu.VMEM((bs * 128,), jnp.float32),
            pltpu.VMEM((bs * 128,), jnp.float32),
            pltpu.VMEM((bs * 128,), jnp.float32),
            pltpu.VMEM((2, 128, H), jnp.float32),
            pltpu.VMEM((16,), jnp.float32),
            pltpu.VMEM_SHARED((ndst,), jnp.float32),
            pltpu.VMEM_SHARED((ndst, H), jnp.float32),
            pltpu.SemaphoreType.DMA,
            pltpu.SemaphoreType.DMA,
            pltpu.SemaphoreType.DMA,
        ],
    )
    def edge_kernel(xs_hbm, asrc_hbm, adst_hbm, src_hbm, dst2_hbm, et_hbm,
                    m_hbm, out_hbm, den_hbm, idx_s, idx_d, et_c, as_c, ad_c,
                    rows, m_v, sh_den, sh_msg, sem, gsem, dsem):
        c = lax.axis_index("c")
        t = lax.axis_index("s")
        w = 2 * t + c

        def _zrow(r, _):
            for f in range(H // 16):
                rows[0, r, pl.ds(f * 16, 16)] = jnp.zeros((16,), jnp.float32)
            return 0
        lax.fori_loop(0, 128, _zrow, 0)

        r0 = t * r16
        nfull, rem = divmod(r16, 128)
        for k in range(nfull):
            pltpu.sync_copy(rows.at[0],
                            sh_msg.at[pl.ds(r0 + k * 128, 128), :])
        if rem:
            pltpu.sync_copy(rows.at[0, pl.ds(0, rem), :],
                            sh_msg.at[pl.ds(r0 + nfull * 128, rem), :])
        nsl = ndst // 128
        for k in range((nsl + 15) // 16):
            i = t + 16 * k

            @pl.when(i < nsl)
            def _zden():
                pltpu.sync_copy(rows.at[0, 0],
                                sh_den.at[pl.ds(i * 128, 128)])

        pltpu.sync_copy(m_hbm.at[pl.ds(0, 16)], m_v)
        plsc.subcore_barrier()
        mb = m_v[...]

        def _batch(b, _):
            e0 = w * slab + b * (bs * 128)
            pltpu.sync_copy(src_hbm.at[pl.ds(e0, bs * 128)], idx_s)
            pltpu.sync_copy(dst2_hbm.at[w, pl.ds(b * bs, bs)], idx_d)
            pltpu.sync_copy(et_hbm.at[pl.ds(e0, bs * 128)], et_c)
            sc_d = []
            for cc in range(bs):
                sl = pl.ds(cc * 128, 128)
                sc_d.append(pltpu.async_copy(
                    asrc_hbm.at[idx_s.at[sl]], as_c.at[sl], sem))
                sc_d.append(pltpu.async_copy(
                    adst_hbm.at[idx_d.at[cc]], ad_c.at[sl], sem))
            for d in sc_d:
                d.wait()
            g = [None, None]
            g[0] = pltpu.async_copy(
                xs_hbm.at[idx_s.at[pl.ds(0, 128)]], rows.at[0], gsem)
            if bs > 1:
                g[1] = pltpu.async_copy(
                    xs_hbm.at[idx_s.at[pl.ds(128, 128)]], rows.at[1], gsem)

            def _exp(i, _):
                sl = pl.ds(i * 16, 16)
                z = as_c[sl] + ad_c[sl] + et_c[sl]
                a = jnp.where(z >= 0.0, z, 0.2 * z)
                et_c[sl] = jnp.exp(a - mb)
                return 0
            lax.fori_loop(0, bs * 8, _exp, 0)

            den_d = [pltpu.async_copy(et_c.at[pl.ds(cc * 128, 128)],
                                      sh_den.at[idx_d.at[cc]], dsem,
                                      add=True)
                     for cc in range(bs)]

            for cc in range(bs):
                pb = cc % 2
                g[pb].wait()

                def _scale(r2, _, cc=cc, pb=pb):
                    for u in range(2):
                        r = 2 * r2 + u
                        cb = plsc.load_gather(
                            et_c, [jnp.full((16,), cc * 128 + r, jnp.int32)])
                        for f in range(H // 16):
                            fl = pl.ds(f * 16, 16)
                            rows[pb, r, fl] = rows[pb, r, fl] * cb
                    return 0
                lax.fori_loop(0, 64, _scale, 0)
                pltpu.sync_copy(rows.at[pb], sh_msg.at[idx_d.at[cc]],
                                add=True)
                nxt = cc + 2
                if nxt < bs:
                    g[pb] = pltpu.async_copy(
                        xs_hbm.at[idx_s.at[pl.ds(nxt * 128, 128)]],
                        rows.at[pb], gsem)
            for d in den_d:
                d.wait()
            return 0
        lax.fori_loop(0, nb, _batch, 0)

        plsc.subcore_barrier()
        pltpu.sync_copy(sh_msg.at[pl.ds(r0, r16), :],
                        out_hbm.at[c, pl.ds(r0, r16), :])
        for k in range((nsl + 15) // 16):
            i = t + 16 * k

            @pl.when(i < nsl)
            def _cden():
                pltpu.sync_copy(sh_den.at[pl.ds(i * 128, 128)],
                                den_hbm.at[c, pl.ds(i * 128, 128)])

    return edge_kernel


_edge_c2c = _make_edge_kernel(NP, NP, EP1, 8)
_edge_c2w = _make_edge_kernel(NP, WP, EP2, 1)


def kernel(cell_x, well_x, c2c_edge_index, c2c_edge_attr, c2w_src, c2w_dst,
           cell_emb_w, cell_emb_b, well_emb_w, well_emb_b, edge_emb_w,
           edge_emb_b, L_lin_w, L_att_src, L_att_dst, L_lin_edge_w,
           L_att_edge, L_gat_bias, L_n1_g, L_n1_b, L_ffn_w1, L_ffn_b1,
           L_ffn_w2, L_ffn_b2, L_n2_g, L_n2_b, wc_lin_w, wc_att_src,
           wc_att_dst, wc_bias, mlp_w1, mlp_b1, mlp_w2, mlp_b2):
    f32 = jnp.float32
    cxp = jnp.zeros((NP, H), f32).at[:N_CELL].set(cell_x)
    wxp = jnp.zeros((WP, well_x.shape[1]), f32).at[:N_WELL].set(well_x)
    attr = c2c_edge_attr[:, 0]

    src1 = jnp.zeros((EP1,), jnp.int32).at[:E_C2C].set(
        c2c_edge_index[0].astype(jnp.int32))
    dst1 = jnp.zeros((EP1,), jnp.int32).at[:E_C2C].set(
        c2c_edge_index[1].astype(jnp.int32))
    dst1_2 = dst1.reshape(NW, EP1 // NW // 128, 128)
    src2 = jnp.zeros((EP2,), jnp.int32).at[:E_C2W].set(
        c2w_src.astype(jnp.int32))
    dst2 = jnp.zeros((EP2,), jnp.int32).at[:E_C2W].set(
        c2w_dst.astype(jnp.int32))
    dst2_2 = dst2.reshape(NW, EP2 // NW // 128, 128)

    h_cell, h_well, et3, etmax3 = _tc(
        _t0_body,
        (jax.ShapeDtypeStruct((NP, H), f32),
         jax.ShapeDtypeStruct((WP, H), f32),
         jax.ShapeDtypeStruct((NL, E_C2C), f32),
         jax.ShapeDtypeStruct((NL, H), f32)))(
        cxp, wxp, attr, cell_emb_w, cell_emb_b, well_emb_w, well_emb_b,
        edge_emb_w, edge_emb_b, L_lin_edge_w, L_att_edge)

    pad_neg = jnp.full((EP1 - E_C2C,), _NEG, f32)
    for l in range(NL):
        xs, a_s, a_d, m = _tc(
            _t1_body,
            (jax.ShapeDtypeStruct((NP, H), f32),
             jax.ShapeDtypeStruct((NP,), f32),
             jax.ShapeDtypeStruct((NP,), f32),
             jax.ShapeDtypeStruct((H,), f32)))(
            h_cell, L_n1_g[l], L_n1_b[l], L_lin_w[l], L_att_src[l],
            L_att_dst[l], etmax3[l])
        et_l = jnp.concatenate([et3[l], pad_neg])
        msg, den = _edge_c2c(xs, a_s, a_d, src1, dst1_2, et_l, m)
        h_cell = _tc(
            _t2_body, jax.ShapeDtypeStruct((NP, H), f32))(
            h_cell, msg[0], msg[1], den[0], den[1], L_gat_bias[l],
            L_n2_g[l], L_n2_b[l], L_ffn_w1[l], L_ffn_b1[l], L_ffn_w2[l],
            L_ffn_b2[l])

    xsw, asw, adw, mw = _tc(
        _t3_body,
        (jax.ShapeDtypeStruct((NP, H), f32),
         jax.ShapeDtypeStruct((NP,), f32),
         jax.ShapeDtypeStruct((WP,), f32),
         jax.ShapeDtypeStruct((H,), f32)))(
        h_cell, h_well, wc_lin_w, wc_att_src, wc_att_dst)
    etw = jnp.concatenate([jnp.zeros((E_C2W,), f32),
                           jnp.full((EP2 - E_C2W,), _NEG, f32)])
    msgw, denw = _edge_c2w(xsw, asw, adw, src2, dst2_2, etw, mw)

    w2p = jnp.zeros((H, H), f32).at[:, :OUT].set(mlp_w2)
    b2p = jnp.zeros((H,), f32).at[:OUT].set(mlp_b2)
    out = _tc(_t4_body, jax.ShapeDtypeStruct((WP, H), f32))(
        msgw[0], msgw[1], denw[0], denw[1], wc_bias, mlp_w1, mlp_b1,
        w2p, b2p)
    return out[:N_WELL, :OUT].reshape(N_WELL, 3, 25)
